# baseline (speedup 1.0000x reference)
import jax
import jax.numpy as jnp
from jax.experimental import pallas as pl

_B, _C, _H, _W = 4, 512, 64, 64
_HW = _H * _W
_TL = 128
_NT = _HW // _TL
_CT = 16
_NC = _NT // _CT


def _rpn_body(x0_ref, xm_ref, xp_ref, wt_ref, s_ref, b3_ref,
              wh_ref, bh_ref, out_ref):
    l = _CT * _TL
    x0 = x0_ref[0].reshape(_C, l)
    xm = xm_ref[0].reshape(_C, l)
    xp = xp_ref[0].reshape(_C, l)
    xs = (xm, x0, xp)

    def z_for(dx):
        k0 = dx + 1
        z = jnp.dot(wt_ref[k0], xs[0], preferred_element_type=jnp.float32)
        z += jnp.dot(wt_ref[k0 + 3], xs[1], preferred_element_type=jnp.float32)
        z += jnp.dot(wt_ref[k0 + 6], xs[2], preferred_element_type=jnp.float32)
        return z

    acc = z_for(0)
    for i, dx in enumerate((-1, 1)):
        zb = z_for(dx).astype(jnp.bfloat16)
        cols = []
        for t in range(_CT):
            sl = slice(t * _TL, (t + 1) * _TL)
            cols.append(jnp.dot(zb[:, sl], s_ref[i],
                                preferred_element_type=jnp.float32))
        acc += jnp.concatenate(cols, axis=1)

    h = jnp.maximum(acc + b3_ref[...], 0.0).astype(jnp.bfloat16)
    o = jnp.dot(wh_ref[...], h, preferred_element_type=jnp.float32)
    out_ref[0] = o + bh_ref[...]


@jax.jit
def kernel(x, conv3_w, conv3_b, reg_w, reg_b, cls_w, cls_b):
    B = x.shape[0]
    xf = x.reshape(B, _C, _HW).astype(jnp.bfloat16)
    x0 = xf.reshape(B, _C, _NT, _TL)
    xm = jnp.pad(xf, ((0, 0), (0, 0), (_H, 0)))[:, :, :_HW]
    xm = xm.reshape(B, _C, _NT, _TL)
    xp = jnp.pad(xf, ((0, 0), (0, 0), (0, _H)))[:, :, _H:]
    xp = xp.reshape(B, _C, _NT, _TL)
    wt = jnp.transpose(conv3_w, (2, 3, 0, 1)).reshape(9, _C, _C)
    wt = wt.astype(jnp.bfloat16)
    ii = jnp.arange(_TL)[:, None]
    jj = jnp.arange(_TL)[None, :]
    s_mats = jnp.stack([
        ((ii == jj + dx) & ((jj % _W) + dx >= 0) & ((jj % _W) + dx < _W))
        .astype(jnp.bfloat16)
        for dx in (-1, 1)])
    wh = jnp.concatenate(
        [reg_w.reshape(36, _C), cls_w.reshape(9, _C),
         jnp.zeros((64 - 45, _C), jnp.float32)], axis=0).astype(jnp.bfloat16)
    bh = jnp.concatenate([reg_b, cls_b, jnp.zeros((64 - 45,), jnp.float32)])

    out = pl.pallas_call(
        _rpn_body,
        grid=(B, _NC),
        in_specs=[
            pl.BlockSpec((1, _C, _CT, _TL), lambda b, c: (b, 0, c, 0)),
            pl.BlockSpec((1, _C, _CT, _TL), lambda b, c: (b, 0, c, 0)),
            pl.BlockSpec((1, _C, _CT, _TL), lambda b, c: (b, 0, c, 0)),
            pl.BlockSpec((9, _C, _C), lambda b, c: (0, 0, 0)),
            pl.BlockSpec((2, _TL, _TL), lambda b, c: (0, 0, 0)),
            pl.BlockSpec((_C, 1), lambda b, c: (0, 0)),
            pl.BlockSpec((64, _C), lambda b, c: (0, 0)),
            pl.BlockSpec((64, 1), lambda b, c: (0, 0)),
        ],
        out_specs=pl.BlockSpec((1, 64, _CT * _TL), lambda b, c: (b, 0, c)),
        out_shape=jax.ShapeDtypeStruct((B, 64, _HW), jnp.float32),
    )(x0, xm, xp, wt, s_mats, conv3_b.reshape(_C, 1), wh, bh.reshape(64, 1))

    ot = jnp.transpose(out, (0, 2, 1))
    box = ot[:, :, :36].reshape(B, _HW * 9, 4)
    cls = ot[:, :, 36:45].reshape(B, _HW * 9, 1)
    return (box, cls)


# speedup vs baseline: 1.1012x; 1.1012x over previous
import jax
import jax.numpy as jnp
from jax.experimental import pallas as pl

_B, _C, _H, _W = 4, 512, 64, 64
_HW = _H * _W
_M = 4096
_R = _HW // _M
_P = 128


def _rpn_body(xs_ref, w9_ref, b3_ref, wh_ref, bh_ref, out_ref):
    r = pl.program_id(1)
    start = r * _M + _P
    acc = jnp.zeros((_M, _C), jnp.float32)
    for k in range(9):
        dy, dx = k // 3 - 1, k % 3 - 1
        src = xs_ref[dx + 1, 0, pl.ds(start + dy * _W, _M), :]
        acc += jnp.dot(src, w9_ref[k], preferred_element_type=jnp.float32)
    h = jnp.maximum(acc + b3_ref[0][None, :], 0.0).astype(jnp.bfloat16)
    o = jnp.dot(h, wh_ref[...], preferred_element_type=jnp.float32)
    out_ref[0] = o + bh_ref[0][None, :]


@jax.jit
def kernel(x, conv3_w, conv3_b, reg_w, reg_b, cls_w, cls_b):
    B = x.shape[0]
    xt = jnp.transpose(x, (0, 2, 3, 1)).reshape(B, _HW, _C)
    xt = xt.astype(jnp.bfloat16)
    z16 = jnp.bfloat16(0)
    wcol = (jnp.arange(_HW) % _W)[None, :, None]
    shifted = []
    for dx in (-1, 0, 1):
        xm = xt
        if dx == -1:
            xm = jnp.where(wcol == _W - 1, z16, xm)
        elif dx == 1:
            xm = jnp.where(wcol == 0, z16, xm)
        xm = jnp.roll(xm, -dx, axis=1) if dx else xm
        shifted.append(jnp.pad(xm, ((0, 0), (_P, _P), (0, 0))))
    xs = jnp.stack(shifted)
    w9 = jnp.transpose(conv3_w, (2, 3, 1, 0)).reshape(9, _C, _C)
    w9 = w9.astype(jnp.bfloat16)
    wh = jnp.concatenate(
        [reg_w.reshape(36, _C).T, cls_w.reshape(9, _C).T,
         jnp.zeros((_C, 64 - 45), jnp.float32)], axis=1).astype(jnp.bfloat16)
    bh = jnp.concatenate([reg_b, cls_b, jnp.zeros((64 - 45,), jnp.float32)])

    out = pl.pallas_call(
        _rpn_body,
        grid=(B, _R),
        in_specs=[
            pl.BlockSpec((3, 1, _HW + 2 * _P, _C), lambda b, r: (0, b, 0, 0)),
            pl.BlockSpec((9, _C, _C), lambda b, r: (0, 0, 0)),
            pl.BlockSpec((1, _C), lambda b, r: (0, 0)),
            pl.BlockSpec((_C, 64), lambda b, r: (0, 0)),
            pl.BlockSpec((1, 64), lambda b, r: (0, 0)),
        ],
        out_specs=pl.BlockSpec((1, _M, 64), lambda b, r: (b, r, 0)),
        out_shape=jax.ShapeDtypeStruct((B, _HW, 64), jnp.float32),
    )(xs, w9, conv3_b.reshape(1, _C), wh, bh.reshape(1, 64))

    box = out[:, :, :36].reshape(B, _HW * 9, 4)
    cls = out[:, :, 36:45].reshape(B, _HW * 9, 1)
    return (box, cls)


# speedup vs baseline: 1.3272x; 1.2052x over previous
import jax
import jax.numpy as jnp
from jax.experimental import pallas as pl
from jax.experimental.pallas import tpu as pltpu

_B, _C, _H, _W = 4, 512, 64, 64
_HW = _H * _W
_PAD = 128
_L = 2048
_NC = _HW // _L


def _rpn_body(x_ref, wt_ref, b3_ref, wh_ref, bh_ref, out_ref):
    wcol = jax.lax.broadcasted_iota(jnp.int32, (1, _L), 1) % _W
    for c in range(_NC):
        base = _PAD + c * _L
        halo = x_ref[0, :, base - _PAD:base + _L + _PAD]
        xs = {}
        xs[0] = halo[:, _PAD:_PAD + _L]
        xs[-1] = pltpu.roll(halo, _H, axis=1)[:, _PAD:_PAD + _L]
        xs[1] = pltpu.roll(halo, _L + 2 * _PAD - _H, axis=1)[:, _PAD:_PAD + _L]

        def z_for(dx):
            k0 = dx + 1
            z = jnp.dot(wt_ref[k0], xs[-1],
                        preferred_element_type=jnp.float32)
            z += jnp.dot(wt_ref[k0 + 3], xs[0],
                         preferred_element_type=jnp.float32)
            z += jnp.dot(wt_ref[k0 + 6], xs[1],
                         preferred_element_type=jnp.float32)
            return z

        acc = z_for(0)
        zm = pltpu.roll(z_for(-1), 1, axis=1)
        acc += jnp.where(wcol == 0, 0.0, zm)
        zp = pltpu.roll(z_for(1), _L - 1, axis=1)
        acc += jnp.where(wcol == _W - 1, 0.0, zp)

        h = jnp.maximum(acc + b3_ref[...], 0.0).astype(jnp.bfloat16)
        o = jnp.dot(wh_ref[...], h, preferred_element_type=jnp.float32)
        out_ref[0, :, c * _L:(c + 1) * _L] = o + bh_ref[...]


@jax.jit
def kernel(x, conv3_w, conv3_b, reg_w, reg_b, cls_w, cls_b):
    B = x.shape[0]
    xf = x.reshape(B, _C, _HW).astype(jnp.bfloat16)
    xpad = jnp.pad(xf, ((0, 0), (0, 0), (_PAD, _PAD)))
    wt = jnp.transpose(conv3_w, (2, 3, 0, 1)).reshape(9, _C, _C)
    wt = wt.astype(jnp.bfloat16)
    wh = jnp.concatenate(
        [reg_w.reshape(36, _C), cls_w.reshape(9, _C),
         jnp.zeros((64 - 45, _C), jnp.float32)], axis=0).astype(jnp.bfloat16)
    bh = jnp.concatenate([reg_b, cls_b, jnp.zeros((64 - 45,), jnp.float32)])

    out = pl.pallas_call(
        _rpn_body,
        grid=(B,),
        in_specs=[
            pl.BlockSpec((1, _C, _HW + 2 * _PAD), lambda b: (b, 0, 0)),
            pl.BlockSpec((9, _C, _C), lambda b: (0, 0, 0)),
            pl.BlockSpec((_C, 1), lambda b: (0, 0)),
            pl.BlockSpec((64, _C), lambda b: (0, 0)),
            pl.BlockSpec((64, 1), lambda b: (0, 0)),
        ],
        out_specs=pl.BlockSpec((1, 64, _HW), lambda b: (b, 0, 0)),
        out_shape=jax.ShapeDtypeStruct((B, 64, _HW), jnp.float32),
    )(xpad, wt, conv3_b.reshape(_C, 1), wh, bh.reshape(64, 1))

    ot = jnp.transpose(out, (0, 2, 1))
    box = ot[:, :, :36].reshape(B, _HW * 9, 4)
    cls = ot[:, :, 36:45].reshape(B, _HW * 9, 1)
    return (box, cls)


# speedup vs baseline: 1.3331x; 1.0044x over previous
import jax
import jax.numpy as jnp
from jax.experimental import pallas as pl
from jax.experimental.pallas import tpu as pltpu

_B, _C, _H, _W = 4, 512, 64, 64
_HW = _H * _W
_PAD = 128
_L = 4096
_NC = _HW // _L


def _rpn_body(x_ref, wt_ref, b3_ref, wh_ref, bh_ref, out_ref):
    wcol = jax.lax.broadcasted_iota(jnp.int32, (1, _L), 1) % _W
    for c in range(_NC):
        base = _PAD + c * _L
        halo = x_ref[0, :, base - _PAD:base + _L + _PAD]
        xs = {}
        xs[0] = halo[:, _PAD:_PAD + _L]
        xs[-1] = pltpu.roll(halo, _H, axis=1)[:, _PAD:_PAD + _L]
        xs[1] = pltpu.roll(halo, _L + 2 * _PAD - _H, axis=1)[:, _PAD:_PAD + _L]

        def z_for(dx):
            k0 = dx + 1
            z = jnp.dot(wt_ref[k0], xs[-1],
                        preferred_element_type=jnp.float32)
            z += jnp.dot(wt_ref[k0 + 3], xs[0],
                         preferred_element_type=jnp.float32)
            z += jnp.dot(wt_ref[k0 + 6], xs[1],
                         preferred_element_type=jnp.float32)
            return z

        acc = z_for(0)
        zm = pltpu.roll(z_for(-1), 1, axis=1)
        acc += jnp.where(wcol == 0, 0.0, zm)
        zp = pltpu.roll(z_for(1), _L - 1, axis=1)
        acc += jnp.where(wcol == _W - 1, 0.0, zp)

        h = jnp.maximum(acc + b3_ref[...], 0.0).astype(jnp.bfloat16)
        o = jnp.dot(wh_ref[...], h, preferred_element_type=jnp.float32)
        out_ref[0, :, c * _L:(c + 1) * _L] = o + bh_ref[...]


@jax.jit
def kernel(x, conv3_w, conv3_b, reg_w, reg_b, cls_w, cls_b):
    B = x.shape[0]
    xf = x.reshape(B, _C, _HW).astype(jnp.bfloat16)
    xpad = jnp.pad(xf, ((0, 0), (0, 0), (_PAD, _PAD)))
    wt = jnp.transpose(conv3_w, (2, 3, 0, 1)).reshape(9, _C, _C)
    wt = wt.astype(jnp.bfloat16)
    wh = jnp.concatenate(
        [reg_w.reshape(36, _C), cls_w.reshape(9, _C),
         jnp.zeros((64 - 45, _C), jnp.float32)], axis=0).astype(jnp.bfloat16)
    bh = jnp.concatenate([reg_b, cls_b, jnp.zeros((64 - 45,), jnp.float32)])

    out = pl.pallas_call(
        _rpn_body,
        grid=(B,),
        in_specs=[
            pl.BlockSpec((1, _C, _HW + 2 * _PAD), lambda b: (b, 0, 0)),
            pl.BlockSpec((9, _C, _C), lambda b: (0, 0, 0)),
            pl.BlockSpec((_C, 1), lambda b: (0, 0)),
            pl.BlockSpec((64, _C), lambda b: (0, 0)),
            pl.BlockSpec((64, 1), lambda b: (0, 0)),
        ],
        out_specs=pl.BlockSpec((1, 64, _HW), lambda b: (b, 0, 0)),
        out_shape=jax.ShapeDtypeStruct((B, 64, _HW), jnp.float32),
    )(xpad, wt, conv3_b.reshape(_C, 1), wh, bh.reshape(64, 1))

    ot = jnp.transpose(out, (0, 2, 1))
    box = ot[:, :, :36].reshape(B, _HW * 9, 4)
    cls = ot[:, :, 36:45].reshape(B, _HW * 9, 1)
    return (box, cls)


# speedup vs baseline: 1.4448x; 1.0838x over previous
import jax
import jax.numpy as jnp
from jax.experimental import pallas as pl
from jax.experimental.pallas import tpu as pltpu

_B, _C, _H, _W = 4, 512, 64, 64
_HW = _H * _W


def _rpn_body(x_ref, wt_ref, b3_ref, wh_ref, bh_ref, out_ref):
    lane = jax.lax.broadcasted_iota(jnp.int32, (1, _HW), 1)
    wcol = lane % _W
    zb = jnp.bfloat16(0)

    xc = x_ref[0].astype(jnp.bfloat16)
    xs = {0: xc}
    xs[-1] = jnp.where(lane < _W, zb, pltpu.roll(xc, _W, axis=1))
    xs[1] = jnp.where(lane >= _HW - _W, zb,
                      pltpu.roll(xc, _HW - _W, axis=1))

    def z_for(dx):
        k0 = dx + 1
        z = jnp.dot(wt_ref[k0], xs[-1], preferred_element_type=jnp.float32)
        z += jnp.dot(wt_ref[k0 + 3], xs[0], preferred_element_type=jnp.float32)
        z += jnp.dot(wt_ref[k0 + 6], xs[1], preferred_element_type=jnp.float32)
        return z

    acc = z_for(0)
    zm = pltpu.roll(z_for(-1), 1, axis=1)
    acc += jnp.where(wcol == 0, 0.0, zm)
    zp = pltpu.roll(z_for(1), _HW - 1, axis=1)
    acc += jnp.where(wcol == _W - 1, 0.0, zp)

    h = jnp.maximum(acc + b3_ref[...], 0.0).astype(jnp.bfloat16)
    o = jnp.dot(wh_ref[...], h, preferred_element_type=jnp.float32)
    out_ref[0] = o + bh_ref[...]


@jax.jit
def kernel(x, conv3_w, conv3_b, reg_w, reg_b, cls_w, cls_b):
    B = x.shape[0]
    xf = x.reshape(B, _C, _HW)
    wt = jnp.transpose(conv3_w, (2, 3, 0, 1)).reshape(9, _C, _C)
    wt = wt.astype(jnp.bfloat16)
    wh = jnp.concatenate(
        [reg_w.reshape(36, _C), cls_w.reshape(9, _C),
         jnp.zeros((64 - 45, _C), jnp.float32)], axis=0).astype(jnp.bfloat16)
    bh = jnp.concatenate([reg_b, cls_b, jnp.zeros((64 - 45,), jnp.float32)])

    out = pl.pallas_call(
        _rpn_body,
        grid=(B,),
        in_specs=[
            pl.BlockSpec((1, _C, _HW), lambda b: (b, 0, 0)),
            pl.BlockSpec((9, _C, _C), lambda b: (0, 0, 0)),
            pl.BlockSpec((_C, 1), lambda b: (0, 0)),
            pl.BlockSpec((64, _C), lambda b: (0, 0)),
            pl.BlockSpec((64, 1), lambda b: (0, 0)),
        ],
        out_specs=pl.BlockSpec((1, 64, _HW), lambda b: (b, 0, 0)),
        out_shape=jax.ShapeDtypeStruct((B, 64, _HW), jnp.float32),
    )(xf, wt, conv3_b.reshape(_C, 1), wh, bh.reshape(64, 1))

    ot = jnp.transpose(out, (0, 2, 1))
    box = ot[:, :, :36].reshape(B, _HW * 9, 4)
    cls = ot[:, :, 36:45].reshape(B, _HW * 9, 1)
    return (box, cls)


# speedup vs baseline: 1.4889x; 1.0305x over previous
import jax
import jax.numpy as jnp
from jax.experimental import pallas as pl
from jax.experimental.pallas import tpu as pltpu

_B, _C, _H, _W = 4, 512, 64, 64
_HW = _H * _W


def _rpn_body(x_ref, wt_ref, b3_ref, wh_ref, bh_ref, box_ref, cls_ref):
    lane = jax.lax.broadcasted_iota(jnp.int32, (1, _HW), 1)
    wcol = lane % _W
    zb = jnp.bfloat16(0)

    xc = x_ref[0].astype(jnp.bfloat16)
    xs = {0: xc}
    xs[-1] = jnp.where(lane < _W, zb, pltpu.roll(xc, _W, axis=1))
    xs[1] = jnp.where(lane >= _HW - _W, zb,
                      pltpu.roll(xc, _HW - _W, axis=1))

    def z_for(dx):
        k0 = dx + 1
        z = jnp.dot(wt_ref[k0], xs[-1], preferred_element_type=jnp.float32)
        z += jnp.dot(wt_ref[k0 + 3], xs[0], preferred_element_type=jnp.float32)
        z += jnp.dot(wt_ref[k0 + 6], xs[1], preferred_element_type=jnp.float32)
        return z

    acc = z_for(0)
    zm = pltpu.roll(z_for(-1), 1, axis=1)
    acc += jnp.where(wcol == 0, 0.0, zm)
    zp = pltpu.roll(z_for(1), _HW - 1, axis=1)
    acc += jnp.where(wcol == _W - 1, 0.0, zp)

    h = jnp.maximum(acc + b3_ref[...], 0.0).astype(jnp.bfloat16)
    o = jnp.dot(wh_ref[...], h, preferred_element_type=jnp.float32)
    ot = jnp.transpose(o) + bh_ref[...]
    box_ref[0] = ot[:, :36]
    cls_ref[0] = ot[:, 36:45]


@jax.jit
def kernel(x, conv3_w, conv3_b, reg_w, reg_b, cls_w, cls_b):
    B = x.shape[0]
    xf = x.reshape(B, _C, _HW)
    wt = jnp.transpose(conv3_w, (2, 3, 0, 1)).reshape(9, _C, _C)
    wt = wt.astype(jnp.bfloat16)
    wh = jnp.concatenate(
        [reg_w.reshape(36, _C), cls_w.reshape(9, _C),
         jnp.zeros((64 - 45, _C), jnp.float32)], axis=0).astype(jnp.bfloat16)
    bh = jnp.concatenate([reg_b, cls_b, jnp.zeros((64 - 45,), jnp.float32)])

    out = pl.pallas_call(
        _rpn_body,
        grid=(B,),
        in_specs=[
            pl.BlockSpec((1, _C, _HW), lambda b: (b, 0, 0)),
            pl.BlockSpec((9, _C, _C), lambda b: (0, 0, 0)),
            pl.BlockSpec((_C, 1), lambda b: (0, 0)),
            pl.BlockSpec((64, _C), lambda b: (0, 0)),
            pl.BlockSpec((1, 64), lambda b: (0, 0)),
        ],
        out_specs=[pl.BlockSpec((1, _HW, 36), lambda b: (b, 0, 0)),
                   pl.BlockSpec((1, _HW, 9), lambda b: (b, 0, 0))],
        out_shape=[jax.ShapeDtypeStruct((B, _HW, 36), jnp.float32),
                   jax.ShapeDtypeStruct((B, _HW, 9), jnp.float32)],
    )(xf, wt, conv3_b.reshape(_C, 1), wh, bh.reshape(1, 64))

    box, cls = out
    return (box.reshape(B, _HW * 9, 4), cls.reshape(B, _HW * 9, 1))
